# speedup vs baseline: 1.5457x; 1.5457x over previous
import functools

import jax
import jax.numpy as jnp
from jax import lax
from jax.experimental import pallas as pl
from jax.experimental.pallas import tpu as pltpu
from jax.experimental.pallas import tpu_sc as plsc

D_MODEL = 1024
B_TOTAL = 4 * 4096

NUM_CORES = 2
NUM_SUBCORES = 16
NUM_WORKERS = NUM_CORES * NUM_SUBCORES
ROWS_PER_WORKER = B_TOTAL // NUM_WORKERS

CHUNK = 64
N_CHUNKS = ROWS_PER_WORKER // CHUNK

_mesh = plsc.VectorSubcoreMesh(core_axis_name="c", subcore_axis_name="s")


@functools.partial(
    pl.kernel,
    mesh=_mesh,
    out_type=jax.ShapeDtypeStruct((B_TOTAL, D_MODEL), jnp.float32),
    scratch_types=[
        pltpu.VMEM((ROWS_PER_WORKER,), jnp.int32),
        pltpu.VMEM((CHUNK, D_MODEL), jnp.float32),
        pltpu.SemaphoreType.DMA,
    ],
)
def _embed_sc(tokens_hbm, table_hbm, out_hbm, idx_v, rows_v, sem):
    wid = lax.axis_index("s") * NUM_CORES + lax.axis_index("c")
    base = wid * ROWS_PER_WORKER
    pltpu.sync_copy(tokens_hbm.at[pl.ds(base, ROWS_PER_WORKER)], idx_v)
    for ci in range(N_CHUNKS):
        off = ci * CHUNK
        pltpu.async_copy(
            table_hbm.at[idx_v.at[pl.ds(off, CHUNK)]], rows_v, sem
        ).wait()
        pltpu.sync_copy(rows_v, out_hbm.at[pl.ds(base + off, CHUNK)])


def kernel(tokens, W_E):
    flat = tokens.reshape(-1).astype(jnp.int32)
    out = _embed_sc(flat, W_E)
    return out.reshape(tokens.shape + (D_MODEL,))


# speedup vs baseline: 1.6244x; 1.0509x over previous
import functools

import jax
import jax.numpy as jnp
from jax import lax
from jax.experimental import pallas as pl
from jax.experimental.pallas import tpu as pltpu
from jax.experimental.pallas import tpu_sc as plsc

D_MODEL = 1024
B_TOTAL = 4 * 4096

NUM_CORES = 2
NUM_SUBCORES = 16
NUM_WORKERS = NUM_CORES * NUM_SUBCORES
ROWS_PER_WORKER = B_TOTAL // NUM_WORKERS

CHUNK = 32
NBUF = 2
N_CHUNKS = ROWS_PER_WORKER // CHUNK

_mesh = plsc.VectorSubcoreMesh(core_axis_name="c", subcore_axis_name="s")


@functools.partial(
    pl.kernel,
    mesh=_mesh,
    out_type=jax.ShapeDtypeStruct((B_TOTAL, D_MODEL), jnp.float32),
    scratch_types=[
        pltpu.VMEM((ROWS_PER_WORKER,), jnp.int32),
        pltpu.VMEM((NBUF, CHUNK, D_MODEL), jnp.float32),
        [pltpu.SemaphoreType.DMA] * NBUF,
        [pltpu.SemaphoreType.DMA] * NBUF,
    ],
)
def _embed_sc(tokens_hbm, table_hbm, out_hbm, idx_v, rows_v, gsems, osems):
    wid = lax.axis_index("s") * NUM_CORES + lax.axis_index("c")
    base = wid * ROWS_PER_WORKER
    pltpu.sync_copy(tokens_hbm.at[pl.ds(base, ROWS_PER_WORKER)], idx_v)

    gathers = [None] * N_CHUNKS
    outs = [None] * N_CHUNKS

    def start_gather(ci):
        b = ci % NBUF
        gathers[ci] = pltpu.async_copy(
            table_hbm.at[idx_v.at[pl.ds(ci * CHUNK, CHUNK)]],
            rows_v.at[b],
            gsems[b],
        )

    def start_out(ci):
        b = ci % NBUF
        outs[ci] = pltpu.async_copy(
            rows_v.at[b],
            out_hbm.at[pl.ds(base + ci * CHUNK, CHUNK)],
            osems[b],
        )

    start_gather(0)
    for ci in range(1, N_CHUNKS):
        if ci >= NBUF:
            outs[ci - NBUF].wait()
        start_gather(ci)
        gathers[ci - 1].wait()
        start_out(ci - 1)
    gathers[N_CHUNKS - 1].wait()
    start_out(N_CHUNKS - 1)
    outs[N_CHUNKS - 2].wait()
    outs[N_CHUNKS - 1].wait()


def kernel(tokens, W_E):
    flat = tokens.reshape(-1).astype(jnp.int32)
    out = _embed_sc(flat, W_E)
    return out.reshape(tokens.shape + (D_MODEL,))


# speedup vs baseline: 1.6354x; 1.0068x over previous
import functools

import jax
import jax.numpy as jnp
from jax import lax
from jax.experimental import pallas as pl
from jax.experimental.pallas import tpu as pltpu
from jax.experimental.pallas import tpu_sc as plsc

D_MODEL = 1024
B_TOTAL = 4 * 4096

NUM_CORES = 2
NUM_SUBCORES = 16
NUM_WORKERS = NUM_CORES * NUM_SUBCORES
ROWS_PER_WORKER = B_TOTAL // NUM_WORKERS

CHUNK = 32
NBUF = 3
N_CHUNKS = ROWS_PER_WORKER // CHUNK

_mesh = plsc.VectorSubcoreMesh(core_axis_name="c", subcore_axis_name="s")


@functools.partial(
    pl.kernel,
    mesh=_mesh,
    out_type=jax.ShapeDtypeStruct((B_TOTAL, D_MODEL), jnp.float32),
    scratch_types=[
        pltpu.VMEM((ROWS_PER_WORKER,), jnp.int32),
        pltpu.VMEM((NBUF, CHUNK, D_MODEL), jnp.float32),
        [pltpu.SemaphoreType.DMA] * NBUF,
        [pltpu.SemaphoreType.DMA] * NBUF,
    ],
)
def _embed_sc(tokens_hbm, table_hbm, out_hbm, idx_v, rows_v, gsems, osems):
    wid = lax.axis_index("s") * NUM_CORES + lax.axis_index("c")
    base = wid * ROWS_PER_WORKER
    pltpu.sync_copy(tokens_hbm.at[pl.ds(base, ROWS_PER_WORKER)], idx_v)

    gathers = [None] * N_CHUNKS
    outs = [None] * N_CHUNKS

    def start_gather(ci):
        b = ci % NBUF
        gathers[ci] = pltpu.async_copy(
            table_hbm.at[idx_v.at[pl.ds(ci * CHUNK, CHUNK)]],
            rows_v.at[b],
            gsems[b],
        )

    def start_out(ci):
        b = ci % NBUF
        outs[ci] = pltpu.async_copy(
            rows_v.at[b],
            out_hbm.at[pl.ds(base + ci * CHUNK, CHUNK)],
            osems[b],
        )

    LAG = NBUF - 1
    for ci in range(N_CHUNKS):
        if ci >= NBUF:
            outs[ci - NBUF].wait()
        start_gather(ci)
        j = ci - LAG
        if j >= 0:
            gathers[j].wait()
            start_out(j)
    for j in range(max(N_CHUNKS - LAG, 0), N_CHUNKS):
        gathers[j].wait()
        start_out(j)
    for j in range(max(N_CHUNKS - NBUF, 0), N_CHUNKS):
        outs[j].wait()


def kernel(tokens, W_E):
    flat = tokens.reshape(-1).astype(jnp.int32)
    out = _embed_sc(flat, W_E)
    return out.reshape(tokens.shape + (D_MODEL,))


# speedup vs baseline: 1.6466x; 1.0068x over previous
import functools

import jax
import jax.numpy as jnp
from jax import lax
from jax.experimental import pallas as pl
from jax.experimental.pallas import tpu as pltpu
from jax.experimental.pallas import tpu_sc as plsc

D_MODEL = 1024
B_TOTAL = 4 * 4096

NUM_CORES = 2
NUM_SUBCORES = 16
NUM_WORKERS = NUM_CORES * NUM_SUBCORES
ROWS_PER_WORKER = B_TOTAL // NUM_WORKERS

CHUNK = 16
NBUF = 7
N_CHUNKS = ROWS_PER_WORKER // CHUNK

_mesh = plsc.VectorSubcoreMesh(core_axis_name="c", subcore_axis_name="s")


@functools.partial(
    pl.kernel,
    mesh=_mesh,
    out_type=jax.ShapeDtypeStruct((B_TOTAL, D_MODEL), jnp.float32),
    scratch_types=[
        pltpu.VMEM((ROWS_PER_WORKER,), jnp.int32),
        pltpu.VMEM((NBUF, CHUNK, D_MODEL), jnp.float32),
        [pltpu.SemaphoreType.DMA] * NBUF,
        [pltpu.SemaphoreType.DMA] * NBUF,
    ],
)
def _embed_sc(tokens_hbm, table_hbm, out_hbm, idx_v, rows_v, gsems, osems):
    wid = lax.axis_index("s") * NUM_CORES + lax.axis_index("c")
    base = wid * ROWS_PER_WORKER
    pltpu.sync_copy(tokens_hbm.at[pl.ds(base, ROWS_PER_WORKER)], idx_v)

    gathers = [None] * N_CHUNKS
    outs = [None] * N_CHUNKS

    def start_gather(ci):
        b = ci % NBUF
        gathers[ci] = pltpu.async_copy(
            table_hbm.at[idx_v.at[pl.ds(ci * CHUNK, CHUNK)]],
            rows_v.at[b],
            gsems[b],
        )

    def start_out(ci):
        b = ci % NBUF
        outs[ci] = pltpu.async_copy(
            rows_v.at[b],
            out_hbm.at[pl.ds(base + ci * CHUNK, CHUNK)],
            osems[b],
        )

    LAG = NBUF - 1
    for ci in range(N_CHUNKS):
        if ci >= NBUF:
            outs[ci - NBUF].wait()
        start_gather(ci)
        j = ci - LAG
        if j >= 0:
            gathers[j].wait()
            start_out(j)
    for j in range(max(N_CHUNKS - LAG, 0), N_CHUNKS):
        gathers[j].wait()
        start_out(j)
    for j in range(max(N_CHUNKS - NBUF, 0), N_CHUNKS):
        outs[j].wait()


def kernel(tokens, W_E):
    flat = tokens.reshape(-1).astype(jnp.int32)
    out = _embed_sc(flat, W_E)
    return out.reshape(tokens.shape + (D_MODEL,))
